# baseline (speedup 1.0000x reference)
import functools

import jax
import jax.numpy as jnp
from jax import lax
from jax.experimental import pallas as pl
from jax.experimental.pallas import tpu as pltpu
from jax.experimental.pallas import tpu_sc as plsc

VOCAB = 1000000
EMBED_DIM = 32
BATCH = 16384
FIELDS = 26

NUM_CORES = 2
NUM_SUBCORES = 16
NUM_WORKERS = NUM_CORES * NUM_SUBCORES

TOTAL_ROWS = BATCH * FIELDS
B_PER_WORKER = BATCH // NUM_WORKERS
CHUNK_B = 32
CHUNK_ROWS = CHUNK_B * FIELDS
NUM_CHUNKS = B_PER_WORKER // CHUNK_B

_mesh = plsc.VectorSubcoreMesh(core_axis_name="c", subcore_axis_name="s")


@functools.partial(
    pl.kernel,
    out_type=jax.ShapeDtypeStruct(
        (FIELDS, EMBED_DIM // 8, BATCH // 128, 8, 128), jnp.float32
    ),
    mesh=_mesh,
    scratch_types=[
        pltpu.VMEM((NUM_CHUNKS, CHUNK_ROWS), jnp.int32),
        pltpu.VMEM((CHUNK_ROWS, EMBED_DIM), jnp.float32),
        pltpu.VMEM((CHUNK_ROWS, EMBED_DIM), jnp.float32),
        pltpu.VMEM((FIELDS, EMBED_DIM // 8, 8, CHUNK_B), jnp.float32),
        pltpu.VMEM((FIELDS, EMBED_DIM // 8, 8, CHUNK_B), jnp.float32),
        pltpu.SemaphoreType.DMA,
        pltpu.SemaphoreType.DMA,
        pltpu.SemaphoreType.DMA,
        pltpu.SemaphoreType.DMA,
    ],
    compiler_params=pltpu.CompilerParams(
        use_tc_tiling_on_sc=False, needs_layout_passes=False
    ),
)
def _sc_gather(
    table_hbm, idx_hbm, out_hbm,
    idx_v, rows0, rows1, stg0, stg1, gsem0, gsem1, wsem0, wsem1,
):
    wid = lax.axis_index("s") * NUM_CORES + lax.axis_index("c")
    pltpu.sync_copy(idx_hbm.at[wid], idx_v)

    rows = (rows0, rows1)
    stg = (stg0, stg1)
    gsems = (gsem0, gsem1)
    wsems = (wsem0, wsem1)

    lane = lax.iota(jnp.int32, 16)
    jbase0 = lane * FIELDS
    jbase1 = (lane + 16) * FIELDS

    gathers = [None, None]
    writes = [None, None]
    gathers[0] = pltpu.async_copy(table_hbm.at[idx_v.at[0]], rows[0], gsem0)

    for i in range(NUM_CHUNKS):
        s = i % 2
        if i + 1 < NUM_CHUNKS:
            gathers[1 - s] = pltpu.async_copy(
                table_hbm.at[idx_v.at[i + 1]], rows[1 - s], gsems[1 - s]
            )
        gathers[s].wait()
        if writes[s] is not None:
            writes[s].wait()

        def transpose_f(f, _, s=s):
            j0 = jbase0 + f
            j1 = jbase1 + f
            for d in range(EMBED_DIM):
                dvec = jnp.full((16,), d, jnp.int32)
                v0 = plsc.load_gather(rows[s], [j0, dvec])
                v1 = plsc.load_gather(rows[s], [j1, dvec])
                stg[s][f, d // 8, d % 8, pl.ds(0, 16)] = v0
                stg[s][f, d // 8, d % 8, pl.ds(16, 16)] = v1
            return _

        lax.fori_loop(0, FIELDS, transpose_f, None)

        bc = wid * (B_PER_WORKER // 128) + i // 4
        blo0 = (i % 4) * CHUNK_B
        writes[s] = pltpu.async_copy(
            stg[s], out_hbm.at[:, :, bc, :, pl.ds(blo0, CHUNK_B)], wsems[s]
        )
    writes[0].wait()
    writes[1].wait()


def kernel(indices, table):
    idx = indices.astype(jnp.int32).reshape(TOTAL_ROWS // 128, 128)
    idx = jax.lax.optimization_barrier(idx)
    idx = idx.reshape(NUM_WORKERS, NUM_CHUNKS, CHUNK_ROWS)
    table2 = table.reshape(VOCAB * EMBED_DIM // 128, 128)
    table2 = jax.lax.optimization_barrier(table2)
    table2 = table2.reshape(VOCAB, EMBED_DIM)
    out5 = _sc_gather(table2, idx)
    return out5.transpose(2, 4, 0, 1, 3).reshape(BATCH, FIELDS, EMBED_DIM)


# speedup vs baseline: 1.0411x; 1.0411x over previous
import functools

import jax
import jax.numpy as jnp
from jax import lax
from jax.experimental import pallas as pl
from jax.experimental.pallas import tpu as pltpu
from jax.experimental.pallas import tpu_sc as plsc

VOCAB = 1000000
EMBED_DIM = 32
BATCH = 16384
FIELDS = 26

NUM_CORES = 2
NUM_SUBCORES = 16
NUM_WORKERS = NUM_CORES * NUM_SUBCORES

TOTAL_ROWS = BATCH * FIELDS
B_PER_WORKER = BATCH // NUM_WORKERS
CHUNK_B = 32
CHUNK_ROWS = CHUNK_B * FIELDS
NUM_CHUNKS = B_PER_WORKER // CHUNK_B

_mesh = plsc.VectorSubcoreMesh(core_axis_name="c", subcore_axis_name="s")


@functools.partial(
    pl.kernel,
    out_type=jax.ShapeDtypeStruct(
        (FIELDS, EMBED_DIM // 8, BATCH // 128, 8, 128), jnp.float32
    ),
    mesh=_mesh,
    scratch_types=[
        pltpu.VMEM((NUM_CHUNKS, CHUNK_ROWS), jnp.int32),
        pltpu.VMEM((CHUNK_ROWS, EMBED_DIM), jnp.float32),
        pltpu.VMEM((CHUNK_ROWS, EMBED_DIM), jnp.float32),
        pltpu.VMEM((FIELDS, EMBED_DIM // 8, 8, CHUNK_B), jnp.float32),
        pltpu.VMEM((FIELDS, EMBED_DIM // 8, 8, CHUNK_B), jnp.float32),
        pltpu.SemaphoreType.DMA,
        pltpu.SemaphoreType.DMA,
        pltpu.SemaphoreType.DMA,
        pltpu.SemaphoreType.DMA,
    ],
    compiler_params=pltpu.CompilerParams(
        use_tc_tiling_on_sc=False, needs_layout_passes=False
    ),
)
def _sc_gather(
    table_hbm, idx_hbm, out_hbm,
    idx_v, rows0, rows1, stg0, stg1, gsem0, gsem1, wsem0, wsem1,
):
    wid = lax.axis_index("s") * NUM_CORES + lax.axis_index("c")
    pltpu.sync_copy(idx_hbm.at[wid], idx_v)

    rows = (rows0, rows1)
    stg = (stg0, stg1)
    gsems = (gsem0, gsem1)
    wsems = (wsem0, wsem1)

    lane = lax.iota(jnp.int32, 16)
    td_lo = lax.shift_right_logical(lane, 3)
    dr_vec = lax.bitwise_and(lane, jnp.int32(7))
    td_hi = td_lo + 2
    f_vecs = [jnp.full((16,), f, jnp.int32) for f in range(FIELDS)]

    gathers = [None, None]
    writes = [None, None]
    gathers[0] = pltpu.async_copy(table_hbm.at[idx_v.at[0]], rows[0], gsem0)

    for i in range(NUM_CHUNKS):
        s = i % 2
        if i + 1 < NUM_CHUNKS:
            gathers[1 - s] = pltpu.async_copy(
                table_hbm.at[idx_v.at[i + 1]], rows[1 - s], gsems[1 - s]
            )
        gathers[s].wait()
        if writes[s] is not None:
            writes[s].wait()

        def transpose_blo(blo, _, s=s):
            blo_vec = jnp.full((16,), 0, jnp.int32) + blo
            jb = blo * FIELDS
            for f in range(FIELDS):
                v0 = rows[s][jb + f, pl.ds(0, 16)]
                v1 = rows[s][jb + f, pl.ds(16, 16)]
                plsc.store_scatter(stg[s], [f_vecs[f], td_lo, dr_vec, blo_vec], v0)
                plsc.store_scatter(stg[s], [f_vecs[f], td_hi, dr_vec, blo_vec], v1)
            return _

        lax.fori_loop(0, CHUNK_B, transpose_blo, None)

        bc = wid * (B_PER_WORKER // 128) + i // 4
        blo0 = (i % 4) * CHUNK_B
        writes[s] = pltpu.async_copy(
            stg[s], out_hbm.at[:, :, bc, :, pl.ds(blo0, CHUNK_B)], wsems[s]
        )
    writes[0].wait()
    writes[1].wait()


def kernel(indices, table):
    idx = indices.astype(jnp.int32).reshape(TOTAL_ROWS // 128, 128)
    idx = jax.lax.optimization_barrier(idx)
    idx = idx.reshape(NUM_WORKERS, NUM_CHUNKS, CHUNK_ROWS)
    table2 = table.reshape(VOCAB * EMBED_DIM // 128, 128)
    table2 = jax.lax.optimization_barrier(table2)
    table2 = table2.reshape(VOCAB, EMBED_DIM)
    out5 = _sc_gather(table2, idx)
    return out5.transpose(2, 4, 0, 1, 3).reshape(BATCH, FIELDS, EMBED_DIM)


# speedup vs baseline: 1.0874x; 1.0445x over previous
import functools

import jax
import jax.numpy as jnp
from jax import lax
from jax.experimental import pallas as pl
from jax.experimental.pallas import tpu as pltpu
from jax.experimental.pallas import tpu_sc as plsc

VOCAB = 1000000
EMBED_DIM = 32
BATCH = 16384
FIELDS = 26

NUM_CORES = 2
NUM_SUBCORES = 16
NUM_WORKERS = NUM_CORES * NUM_SUBCORES

TOTAL_ROWS = BATCH * FIELDS
B_PER_WORKER = BATCH // NUM_WORKERS
CHUNK_B = 32
CHUNK_ROWS = CHUNK_B * FIELDS
NUM_CHUNKS = B_PER_WORKER // CHUNK_B

_mesh = plsc.VectorSubcoreMesh(core_axis_name="c", subcore_axis_name="s")


@functools.partial(
    pl.kernel,
    out_type=jax.ShapeDtypeStruct(
        (FIELDS, EMBED_DIM // 8, BATCH // 128, 8, 128), jnp.float32
    ),
    mesh=_mesh,
    scratch_types=[
        pltpu.VMEM((NUM_CHUNKS, CHUNK_ROWS), jnp.int32),
        pltpu.VMEM((CHUNK_ROWS, EMBED_DIM), jnp.float32),
        pltpu.VMEM((CHUNK_ROWS, EMBED_DIM), jnp.float32),
        pltpu.VMEM((FIELDS, EMBED_DIM // 8, 8, CHUNK_B), jnp.float32),
        pltpu.VMEM((FIELDS, EMBED_DIM // 8, 8, CHUNK_B), jnp.float32),
        pltpu.SemaphoreType.DMA,
        pltpu.SemaphoreType.DMA,
        pltpu.SemaphoreType.DMA,
        pltpu.SemaphoreType.DMA,
    ],
    compiler_params=pltpu.CompilerParams(
        use_tc_tiling_on_sc=False, needs_layout_passes=False
    ),
)
def _sc_gather(
    table_hbm, idx_hbm, out_hbm,
    idx_v, rows0, rows1, stg0, stg1, gsem0, gsem1, wsem0, wsem1,
):
    wid = lax.axis_index("s") * NUM_CORES + lax.axis_index("c")
    pltpu.sync_copy(idx_hbm.at[wid], idx_v)

    rows = (rows0, rows1)
    stg = (stg0, stg1)
    gsems = (gsem0, gsem1)
    wsems = (wsem0, wsem1)

    lane = lax.iota(jnp.int32, 16)
    td_lo = lax.shift_right_logical(lane, 3)
    dr_vec = lax.bitwise_and(lane, jnp.int32(7))
    td_hi = td_lo + 2
    f_vecs = [jnp.full((16,), f, jnp.int32) for f in range(FIELDS)]

    gathers = [None, None]
    writes = [None, None]
    gathers[0] = pltpu.async_copy(table_hbm.at[idx_v.at[0]], rows[0], gsem0)

    for i in range(NUM_CHUNKS):
        s = i % 2
        if i + 1 < NUM_CHUNKS:
            gathers[1 - s] = pltpu.async_copy(
                table_hbm.at[idx_v.at[i + 1]], rows[1 - s], gsems[1 - s]
            )
        gathers[s].wait()
        if writes[s] is not None:
            writes[s].wait()

        @plsc.parallel_loop(0, CHUNK_B, unroll=4)
        def transpose_blo(blo, s=s):
            blo_vec = jnp.full((16,), 0, jnp.int32) + blo
            jb = blo * FIELDS
            for f in range(FIELDS):
                v0 = rows[s][jb + f, pl.ds(0, 16)]
                v1 = rows[s][jb + f, pl.ds(16, 16)]
                plsc.store_scatter(stg[s], [f_vecs[f], td_lo, dr_vec, blo_vec], v0)
                plsc.store_scatter(stg[s], [f_vecs[f], td_hi, dr_vec, blo_vec], v1)

        bc = wid * (B_PER_WORKER // 128) + i // 4
        blo0 = (i % 4) * CHUNK_B
        writes[s] = pltpu.async_copy(
            stg[s], out_hbm.at[:, :, bc, :, pl.ds(blo0, CHUNK_B)], wsems[s]
        )
    writes[0].wait()
    writes[1].wait()


def kernel(indices, table):
    idx = indices.astype(jnp.int32).reshape(TOTAL_ROWS // 128, 128)
    idx = jax.lax.optimization_barrier(idx)
    idx = idx.reshape(NUM_WORKERS, NUM_CHUNKS, CHUNK_ROWS)
    table2 = table.reshape(VOCAB * EMBED_DIM // 128, 128)
    table2 = jax.lax.optimization_barrier(table2)
    table2 = table2.reshape(VOCAB, EMBED_DIM)
    out5 = _sc_gather(table2, idx)
    return out5.transpose(2, 4, 0, 1, 3).reshape(BATCH, FIELDS, EMBED_DIM)


# speedup vs baseline: 1.0885x; 1.0010x over previous
import functools

import jax
import jax.numpy as jnp
from jax import lax
from jax.experimental import pallas as pl
from jax.experimental.pallas import tpu as pltpu
from jax.experimental.pallas import tpu_sc as plsc

VOCAB = 1000000
EMBED_DIM = 32
BATCH = 16384
FIELDS = 26

NUM_CORES = 2
NUM_SUBCORES = 16
NUM_WORKERS = NUM_CORES * NUM_SUBCORES

TOTAL_ROWS = BATCH * FIELDS
B_PER_WORKER = BATCH // NUM_WORKERS
CHUNK_B = 32
CHUNK_ROWS = CHUNK_B * FIELDS
NUM_CHUNKS = B_PER_WORKER // CHUNK_B

_mesh = plsc.VectorSubcoreMesh(core_axis_name="c", subcore_axis_name="s")


@functools.partial(
    pl.kernel,
    out_type=jax.ShapeDtypeStruct(
        (FIELDS, EMBED_DIM // 8, BATCH // 128, 8, 128), jnp.float32
    ),
    mesh=_mesh,
    scratch_types=[
        pltpu.VMEM((NUM_CHUNKS, CHUNK_ROWS), jnp.int32),
        pltpu.VMEM((CHUNK_ROWS, EMBED_DIM), jnp.float32),
        pltpu.VMEM((CHUNK_ROWS, EMBED_DIM), jnp.float32),
        pltpu.VMEM((FIELDS, EMBED_DIM // 8, 8, CHUNK_B), jnp.float32),
        pltpu.VMEM((FIELDS, EMBED_DIM // 8, 8, CHUNK_B), jnp.float32),
        pltpu.SemaphoreType.DMA,
        pltpu.SemaphoreType.DMA,
        pltpu.SemaphoreType.DMA,
        pltpu.SemaphoreType.DMA,
    ],
    compiler_params=pltpu.CompilerParams(
        use_tc_tiling_on_sc=False, needs_layout_passes=False
    ),
)
def _sc_gather(
    table_hbm, idx_hbm, out_hbm,
    idx_v, rows0, rows1, stg0, stg1, gsem0, gsem1, wsem0, wsem1,
):
    wid = lax.axis_index("s") * NUM_CORES + lax.axis_index("c")
    pltpu.sync_copy(idx_hbm.at[wid], idx_v)

    rows = (rows0, rows1)
    stg = (stg0, stg1)
    gsems = (gsem0, gsem1)
    wsems = (wsem0, wsem1)

    lane = lax.iota(jnp.int32, 16)
    td_lo = lax.shift_right_logical(lane, 3)
    dr_vec = lax.bitwise_and(lane, jnp.int32(7))
    td_hi = td_lo + 2

    gathers = [None, None]
    writes = [None, None]
    gathers[0] = pltpu.async_copy(table_hbm.at[idx_v.at[0]], rows[0], gsem0)

    for i in range(NUM_CHUNKS):
        s = i % 2
        if i + 1 < NUM_CHUNKS:
            gathers[1 - s] = pltpu.async_copy(
                table_hbm.at[idx_v.at[i + 1]], rows[1 - s], gsems[1 - s]
            )
        gathers[s].wait()
        if writes[s] is not None:
            writes[s].wait()

        @plsc.parallel_loop(0, CHUNK_B, unroll=4)
        def transpose_blo(blo, s=s):
            blo_vec = jnp.full((16,), 0, jnp.int32) + blo
            jb = blo * FIELDS
            for f in range(FIELDS):
                v0 = rows[s][jb + f, pl.ds(0, 16)]
                v1 = rows[s][jb + f, pl.ds(16, 16)]
                plsc.store_scatter(stg[s].at[f], [td_lo, dr_vec, blo_vec], v0)
                plsc.store_scatter(stg[s].at[f], [td_hi, dr_vec, blo_vec], v1)

        bc = wid * (B_PER_WORKER // 128) + i // 4
        blo0 = (i % 4) * CHUNK_B
        writes[s] = pltpu.async_copy(
            stg[s], out_hbm.at[:, :, bc, :, pl.ds(blo0, CHUNK_B)], wsems[s]
        )
    writes[0].wait()
    writes[1].wait()


def kernel(indices, table):
    idx = indices.astype(jnp.int32).reshape(TOTAL_ROWS // 128, 128)
    idx = jax.lax.optimization_barrier(idx)
    idx = idx.reshape(NUM_WORKERS, NUM_CHUNKS, CHUNK_ROWS)
    table2 = table.reshape(VOCAB * EMBED_DIM // 128, 128)
    table2 = jax.lax.optimization_barrier(table2)
    table2 = table2.reshape(VOCAB, EMBED_DIM)
    out5 = _sc_gather(table2, idx)
    return out5.transpose(2, 4, 0, 1, 3).reshape(BATCH, FIELDS, EMBED_DIM)


# speedup vs baseline: 1.4319x; 1.3154x over previous
import functools

import jax
import jax.numpy as jnp
from jax import lax
from jax.experimental import pallas as pl
from jax.experimental.pallas import tpu as pltpu
from jax.experimental.pallas import tpu_sc as plsc

VOCAB = 1000000
EMBED_DIM = 32
BATCH = 16384
FIELDS = 26

NUM_CORES = 2
NUM_SUBCORES = 16
NUM_WORKERS = NUM_CORES * NUM_SUBCORES

TOTAL_ROWS = BATCH * FIELDS
B_PER_WORKER = BATCH // NUM_WORKERS
CHUNK_B = 32
CHUNK_ROWS = CHUNK_B * FIELDS
NUM_CHUNKS = B_PER_WORKER // CHUNK_B

_mesh = plsc.VectorSubcoreMesh(core_axis_name="c", subcore_axis_name="s")


@functools.partial(
    pl.kernel,
    out_type=jax.ShapeDtypeStruct(
        (FIELDS, EMBED_DIM // 8, BATCH // 128, 8, 128), jnp.float32
    ),
    mesh=_mesh,
    scratch_types=[
        pltpu.VMEM((NUM_CHUNKS, CHUNK_ROWS), jnp.int32),
        pltpu.VMEM((CHUNK_ROWS, EMBED_DIM), jnp.float32),
        pltpu.VMEM((CHUNK_ROWS, EMBED_DIM), jnp.float32),
        pltpu.VMEM((FIELDS, EMBED_DIM // 8, 8, CHUNK_B + 1), jnp.float32),
        pltpu.SemaphoreType.DMA,
        pltpu.SemaphoreType.DMA,
        pltpu.SemaphoreType.DMA,
    ],
    compiler_params=pltpu.CompilerParams(
        use_tc_tiling_on_sc=False, needs_layout_passes=False
    ),
)
def _sc_gather(
    table_hbm, idx_hbm, out_hbm,
    idx_v, rows0, rows1, stg0, gsem0, gsem1, wsem0,
):
    wid = lax.axis_index("s") * NUM_CORES + lax.axis_index("c")
    pltpu.sync_copy(idx_hbm.at[wid], idx_v)

    rows = (rows0, rows1)
    stg = stg0
    gsems = (gsem0, gsem1)

    lane = lax.iota(jnp.int32, 16)
    td_lo = lax.shift_right_logical(lane, 3)
    dr_vec = lax.bitwise_and(lane, jnp.int32(7))
    td_hi = td_lo + 2

    gathers = [None, None]
    write = [None]
    gathers[0] = pltpu.async_copy(table_hbm.at[idx_v.at[0]], rows[0], gsem0)

    for i in range(NUM_CHUNKS):
        s = i % 2
        if i + 1 < NUM_CHUNKS:
            gathers[1 - s] = pltpu.async_copy(
                table_hbm.at[idx_v.at[i + 1]], rows[1 - s], gsems[1 - s]
            )
        gathers[s].wait()
        if write[0] is not None:
            write[0].wait()

        @plsc.parallel_loop(0, CHUNK_B, unroll=4)
        def transpose_blo(blo, s=s):
            blo_vec = jnp.full((16,), 0, jnp.int32) + blo
            jb = blo * FIELDS
            for f in range(FIELDS):
                v0 = rows[s][jb + f, pl.ds(0, 16)]
                v1 = rows[s][jb + f, pl.ds(16, 16)]
                plsc.store_scatter(stg.at[f], [td_lo, dr_vec, blo_vec], v0)
                plsc.store_scatter(stg.at[f], [td_hi, dr_vec, blo_vec], v1)

        bc = wid * (B_PER_WORKER // 128) + i // 4
        blo0 = (i % 4) * CHUNK_B
        write[0] = pltpu.async_copy(
            stg.at[:, :, :, pl.ds(0, CHUNK_B)],
            out_hbm.at[:, :, bc, :, pl.ds(blo0, CHUNK_B)],
            wsem0,
        )
    write[0].wait()


def kernel(indices, table):
    idx = indices.astype(jnp.int32).reshape(TOTAL_ROWS // 128, 128)
    idx = jax.lax.optimization_barrier(idx)
    idx = idx.reshape(NUM_WORKERS, NUM_CHUNKS, CHUNK_ROWS)
    table2 = table.reshape(VOCAB * EMBED_DIM // 128, 128)
    table2 = jax.lax.optimization_barrier(table2)
    table2 = table2.reshape(VOCAB, EMBED_DIM)
    out5 = _sc_gather(table2, idx)
    return out5.transpose(2, 4, 0, 1, 3).reshape(BATCH, FIELDS, EMBED_DIM)
